# baseline (speedup 1.0000x reference)
import functools

import jax
import jax.numpy as jnp
from jax import lax
from jax.experimental import pallas as pl
from jax.experimental.pallas import tpu as pltpu
from jax.experimental.pallas import tpu_sc as plsc

D = 768
E = 64
H = 1536
N = 16384
TB = 512
NB = N // TB
T = 128
NT = N // T + E - 1
NTP = 192
NROWS = NT * T

SC_CORES = 2
SC_SUBCORES = 16
NW = SC_CORES * SC_SUBCORES
ROWS_PER_W = N // NW
CH = 64
NCH = ROWS_PER_W // CH


def _router_body(x_ref, wg_ref, bg_ref, gate_ref, e_ref, rank_ref,
                 counts_ref, run_ref):
    i = pl.program_id(0)

    @pl.when(i == 0)
    def _():
        run_ref[...] = jnp.zeros_like(run_ref)

    xb = x_ref[...]
    logits = jnp.dot(xb, wg_ref[...], preferred_element_type=jnp.float32)
    logits = logits + bg_ref[...]
    m = jnp.max(logits, axis=1, keepdims=True)
    lane = lax.broadcasted_iota(jnp.int32, (TB, E), 1)
    idx = jnp.min(jnp.where(logits == m, lane, E), axis=1, keepdims=True)
    oh = (lane == idx).astype(jnp.float32)
    gate_ref[...] = oh
    e_ref[...] = idx
    tri = (lax.broadcasted_iota(jnp.int32, (TB, TB), 0)
           > lax.broadcasted_iota(jnp.int32, (TB, TB), 1)).astype(jnp.float32)
    excl = jnp.dot(tri, oh, preferred_element_type=jnp.float32) + run_ref[...]
    rank_ref[...] = jnp.sum(excl * oh, axis=1, keepdims=True).astype(jnp.int32)
    run_ref[...] = run_ref[...] + jnp.sum(oh, axis=0, keepdims=True)
    counts_ref[...] = run_ref[...]


def _router(x_flat, Wg, bg):
    return pl.pallas_call(
        _router_body,
        grid=(NB,),
        in_specs=[
            pl.BlockSpec((TB, D), lambda i: (i, 0)),
            pl.BlockSpec((D, E), lambda i: (0, 0)),
            pl.BlockSpec((1, E), lambda i: (0, 0)),
        ],
        out_specs=[
            pl.BlockSpec((TB, E), lambda i: (i, 0)),
            pl.BlockSpec((TB, 1), lambda i: (i, 0)),
            pl.BlockSpec((TB, 1), lambda i: (i, 0)),
            pl.BlockSpec((1, E), lambda i: (0, 0)),
        ],
        out_shape=[
            jax.ShapeDtypeStruct((N, E), jnp.float32),
            jax.ShapeDtypeStruct((N, 1), jnp.int32),
            jax.ShapeDtypeStruct((N, 1), jnp.int32),
            jax.ShapeDtypeStruct((1, E), jnp.float32),
        ],
        scratch_shapes=[pltpu.VMEM((1, E), jnp.float32)],
    )(x_flat, Wg, bg.reshape(1, E))


def _plan_body(counts_ref, gate_ref, rank_ref, p_ref, te_ref, ntu_ref):
    tiles = (counts_ref[...].astype(jnp.int32) + (T - 1)) // T

    def shr(v, k):
        return jnp.concatenate([jnp.zeros((1, k), v.dtype), v[:, :E - k]],
                               axis=1)

    s = shr(tiles, 1)
    for k in (1, 2, 4, 8, 16, 32):
        s = s + shr(s, k)
    starts = s * T

    st = jnp.sum(gate_ref[...] * starts.astype(jnp.float32), axis=1,
                 keepdims=True)
    p_ref[...] = rank_ref[...] + st.astype(jnp.int32)

    slot = lax.broadcasted_iota(jnp.int32, (NTP, E), 0) * T
    te_ref[...] = jnp.sum((starts <= slot).astype(jnp.int32), axis=1,
                          keepdims=True) - 1
    ntu_ref[...] = jnp.sum(tiles, axis=1, keepdims=True)


def _plan(counts, gate, rank):
    return pl.pallas_call(
        _plan_body,
        grid=(NB,),
        in_specs=[
            pl.BlockSpec((1, E), lambda i: (0, 0)),
            pl.BlockSpec((TB, E), lambda i: (i, 0)),
            pl.BlockSpec((TB, 1), lambda i: (i, 0)),
        ],
        out_specs=[
            pl.BlockSpec((TB, 1), lambda i: (i, 0)),
            pl.BlockSpec((NTP, 1), lambda i: (0, 0)),
            pl.BlockSpec((1, 1), lambda i: (0, 0)),
        ],
        out_shape=[
            jax.ShapeDtypeStruct((N, 1), jnp.int32),
            jax.ShapeDtypeStruct((NTP, 1), jnp.int32),
            jax.ShapeDtypeStruct((1, 1), jnp.int32),
        ],
    )(counts, gate, rank)


@functools.cache
def _sc_kernels():
    mesh = plsc.VectorSubcoreMesh(core_axis_name="c", subcore_axis_name="s",
                                  num_cores=SC_CORES, num_subcores=SC_SUBCORES)
    scratch = [
        pltpu.VMEM((CH,), jnp.int32),
        pltpu.VMEM((CH,), jnp.int32),
        pltpu.VMEM((CH, D), jnp.float32),
        pltpu.VMEM((CH, D), jnp.float32),
        pltpu.SemaphoreType.DMA,
        pltpu.SemaphoreType.DMA,
        pltpu.SemaphoreType.DMA,
    ]

    @functools.partial(
        pl.kernel,
        out_type=jax.ShapeDtypeStruct((NROWS, D), jnp.float32),
        mesh=mesh,
        scratch_types=scratch,
    )
    def scatter(x_hbm, p_hbm, xs_hbm, idx0, idx1, rows0, rows1, sem0, sem1,
                sem_w):
        wid = lax.axis_index("s") * SC_CORES + lax.axis_index("c")
        base = wid * ROWS_PER_W
        idx = (idx0, idx1)
        rows = (rows0, rows1)
        sems = (sem0, sem1)
        pltpu.async_copy(p_hbm.at[pl.ds(base, CH)], idx0, sem0)
        pltpu.async_copy(x_hbm.at[pl.ds(base, CH), :], rows0, sem0)
        for c in range(NCH):
            b = c % 2
            nb = (c + 1) % 2
            if c + 1 < NCH:
                off = base + (c + 1) * CH
                pltpu.async_copy(p_hbm.at[pl.ds(off, CH)], idx[nb], sems[nb])
                pltpu.async_copy(x_hbm.at[pl.ds(off, CH), :], rows[nb],
                                 sems[nb])
            pltpu.make_async_copy(p_hbm.at[pl.ds(base, CH)], idx[b],
                                  sems[b]).wait()
            pltpu.make_async_copy(x_hbm.at[pl.ds(base, CH), :], rows[b],
                                  sems[b]).wait()
            pltpu.async_copy(rows[b], xs_hbm.at[idx[b]], sem_w).wait()

    @functools.partial(
        pl.kernel,
        out_type=jax.ShapeDtypeStruct((N, D), jnp.float32),
        mesh=mesh,
        scratch_types=scratch,
    )
    def gather(ys_hbm, p_hbm, out_hbm, idx0, idx1, rows0, rows1, sem0, sem1,
               sem_w):
        wid = lax.axis_index("s") * SC_CORES + lax.axis_index("c")
        base = wid * ROWS_PER_W
        idx = (idx0, idx1)
        rows = (rows0, rows1)
        sems = (sem0, sem1)
        pltpu.async_copy(p_hbm.at[pl.ds(base, CH)], idx0, sem0)
        for c in range(NCH):
            b = c % 2
            nb = (c + 1) % 2
            if c + 1 < NCH:
                off = base + (c + 1) * CH
                pltpu.async_copy(p_hbm.at[pl.ds(off, CH)], idx[nb], sems[nb])
            pltpu.make_async_copy(p_hbm.at[pl.ds(base, CH)], idx[b],
                                  sems[b]).wait()
            pltpu.async_copy(ys_hbm.at[idx[b]], rows[b], sem_w).wait()
            pltpu.sync_copy(rows[b], out_hbm.at[pl.ds(base + c * CH, CH), :])

    return scatter, gather


def _sc_scatter(x_flat, p_flat):
    return _sc_kernels()[0](x_flat, p_flat)


def _sc_gather(ys, p_flat):
    return _sc_kernels()[1](ys, p_flat)


S1 = 2
S2 = 2
D1 = D // S1
H1 = H // S2


def _ffn_body(te_ref, ntu_ref, xs_ref, w1a_ref, w1b_ref, b1_ref, w2a_ref,
              w2b_ref, b2_ref, ys_ref):
    @pl.when(pl.program_id(0) < ntu_ref[0])
    def _():
        xb = xs_ref[...].astype(jnp.bfloat16)
        h = jnp.dot(xb[:, :D1], w1a_ref[0].astype(jnp.bfloat16),
                    preferred_element_type=jnp.float32)
        h = h + jnp.dot(xb[:, D1:], w1b_ref[0].astype(jnp.bfloat16),
                        preferred_element_type=jnp.float32)
        h = jnp.maximum(h + b1_ref[0], 0.0).astype(jnp.bfloat16)
        y = jnp.dot(h[:, :H1], w2a_ref[0].astype(jnp.bfloat16),
                    preferred_element_type=jnp.float32)
        y = y + jnp.dot(h[:, H1:], w2b_ref[0].astype(jnp.bfloat16),
                        preferred_element_type=jnp.float32)
        ys_ref[...] = y + b2_ref[0]


def _ffn(te, ntu, xs, W1, b1, W2, b2):
    def _clamp(i, ntu):
        return jnp.minimum(i, ntu[0] - 1)

    grid_spec = pltpu.PrefetchScalarGridSpec(
        num_scalar_prefetch=2,
        grid=(NT,),
        in_specs=[
            pl.BlockSpec((T, D), lambda i, te, ntu: (_clamp(i, ntu), 0)),
            pl.BlockSpec((1, D1, H),
                         lambda i, te, ntu: (te[_clamp(i, ntu)], 0, 0)),
            pl.BlockSpec((1, D1, H),
                         lambda i, te, ntu: (te[_clamp(i, ntu)], 1, 0)),
            pl.BlockSpec((1, 1, H),
                         lambda i, te, ntu: (te[_clamp(i, ntu)], 0, 0)),
            pl.BlockSpec((1, H1, D),
                         lambda i, te, ntu: (te[_clamp(i, ntu)], 0, 0)),
            pl.BlockSpec((1, H1, D),
                         lambda i, te, ntu: (te[_clamp(i, ntu)], 1, 0)),
            pl.BlockSpec((1, 1, D),
                         lambda i, te, ntu: (te[_clamp(i, ntu)], 0, 0)),
        ],
        out_specs=pl.BlockSpec((T, D), lambda i, te, ntu: (_clamp(i, ntu), 0)),
    )
    return pl.pallas_call(
        _ffn_body,
        grid_spec=grid_spec,
        out_shape=jax.ShapeDtypeStruct((NROWS, D), jnp.float32),
    )(te, ntu, xs, W1, W1, b1.reshape(E, 1, H), W2, W2,
      b2.reshape(E, 1, D))


def kernel(x, Wg, bg, W1, b1, W2, b2):
    Bb, Ss, _ = x.shape
    x_flat = x.reshape(-1, D)
    gate, ecol, rank, counts = _router(x_flat, Wg, bg)
    p, te, ntu = _plan(counts, gate, rank)
    p_flat = p.reshape(-1)
    xs = _sc_scatter(x_flat, p_flat)
    ys = _ffn(te.reshape(-1), ntu.reshape(-1), xs, W1, b1, W2, b2)
    out = _sc_gather(ys, p_flat)
    return out.reshape(Bb, Ss, D), gate


# speedup vs baseline: 1.2030x; 1.2030x over previous
import functools

import jax
import jax.numpy as jnp
from jax import lax
from jax.experimental import pallas as pl
from jax.experimental.pallas import tpu as pltpu
from jax.experimental.pallas import tpu_sc as plsc

D = 768
E = 64
H = 1536
N = 16384
TB = 512
NB = N // TB
T = 256
NT = N // T + E - 1
NTP = NT + 1
NROWS = NT * T

SC_CORES = 2
SC_SUBCORES = 16
NW = SC_CORES * SC_SUBCORES
ROWS_PER_W = N // NW
CH = 64
NCH = ROWS_PER_W // CH


def _router_body(x_ref, wg_ref, bg_ref, gate_ref, e_ref, rank_ref,
                 counts_ref, run_ref):
    i = pl.program_id(0)

    @pl.when(i == 0)
    def _():
        run_ref[...] = jnp.zeros_like(run_ref)

    xb = x_ref[...]
    logits = jnp.dot(xb, wg_ref[...], preferred_element_type=jnp.float32)
    logits = logits + bg_ref[...]
    m = jnp.max(logits, axis=1, keepdims=True)
    lane = lax.broadcasted_iota(jnp.int32, (TB, E), 1)
    idx = jnp.min(jnp.where(logits == m, lane, E), axis=1, keepdims=True)
    oh = (lane == idx).astype(jnp.float32)
    gate_ref[...] = oh
    e_ref[...] = idx
    tri = (lax.broadcasted_iota(jnp.int32, (TB, TB), 0)
           > lax.broadcasted_iota(jnp.int32, (TB, TB), 1)).astype(jnp.float32)
    excl = jnp.dot(tri, oh, preferred_element_type=jnp.float32) + run_ref[...]
    rank_ref[...] = jnp.sum(excl * oh, axis=1, keepdims=True).astype(jnp.int32)
    run_ref[...] = run_ref[...] + jnp.sum(oh, axis=0, keepdims=True)
    counts_ref[...] = run_ref[...]


def _router(x_flat, Wg, bg):
    return pl.pallas_call(
        _router_body,
        grid=(NB,),
        in_specs=[
            pl.BlockSpec((TB, D), lambda i: (i, 0)),
            pl.BlockSpec((D, E), lambda i: (0, 0)),
            pl.BlockSpec((1, E), lambda i: (0, 0)),
        ],
        out_specs=[
            pl.BlockSpec((TB, E), lambda i: (i, 0)),
            pl.BlockSpec((TB, 1), lambda i: (i, 0)),
            pl.BlockSpec((TB, 1), lambda i: (i, 0)),
            pl.BlockSpec((1, E), lambda i: (0, 0)),
        ],
        out_shape=[
            jax.ShapeDtypeStruct((N, E), jnp.float32),
            jax.ShapeDtypeStruct((N, 1), jnp.int32),
            jax.ShapeDtypeStruct((N, 1), jnp.int32),
            jax.ShapeDtypeStruct((1, E), jnp.float32),
        ],
        scratch_shapes=[pltpu.VMEM((1, E), jnp.float32)],
    )(x_flat, Wg, bg.reshape(1, E))


def _plan_body(counts_ref, gate_ref, rank_ref, p_ref, te_ref, ntu_ref):
    tiles = (counts_ref[...].astype(jnp.int32) + (T - 1)) // T

    def shr(v, k):
        return jnp.concatenate([jnp.zeros((1, k), v.dtype), v[:, :E - k]],
                               axis=1)

    s = shr(tiles, 1)
    for k in (1, 2, 4, 8, 16, 32):
        s = s + shr(s, k)
    starts = s * T

    st = jnp.sum(gate_ref[...] * starts.astype(jnp.float32), axis=1,
                 keepdims=True)
    p_ref[...] = rank_ref[...] + st.astype(jnp.int32)

    slot = lax.broadcasted_iota(jnp.int32, (NTP, E), 0) * T
    te_ref[...] = jnp.sum((starts <= slot).astype(jnp.int32), axis=1,
                          keepdims=True) - 1
    ntu_ref[...] = jnp.sum(tiles, axis=1, keepdims=True)


def _plan(counts, gate, rank):
    return pl.pallas_call(
        _plan_body,
        grid=(NB,),
        in_specs=[
            pl.BlockSpec((1, E), lambda i: (0, 0)),
            pl.BlockSpec((TB, E), lambda i: (i, 0)),
            pl.BlockSpec((TB, 1), lambda i: (i, 0)),
        ],
        out_specs=[
            pl.BlockSpec((TB, 1), lambda i: (i, 0)),
            pl.BlockSpec((NTP, 1), lambda i: (0, 0)),
            pl.BlockSpec((1, 1), lambda i: (0, 0)),
        ],
        out_shape=[
            jax.ShapeDtypeStruct((N, 1), jnp.int32),
            jax.ShapeDtypeStruct((NTP, 1), jnp.int32),
            jax.ShapeDtypeStruct((1, 1), jnp.int32),
        ],
    )(counts, gate, rank)


@functools.cache
def _sc_kernels():
    mesh = plsc.VectorSubcoreMesh(core_axis_name="c", subcore_axis_name="s",
                                  num_cores=SC_CORES, num_subcores=SC_SUBCORES)
    scratch = [
        pltpu.VMEM((CH,), jnp.int32),
        pltpu.VMEM((CH,), jnp.int32),
        pltpu.VMEM((CH, D), jnp.float32),
        pltpu.VMEM((CH, D), jnp.float32),
        pltpu.SemaphoreType.DMA,
        pltpu.SemaphoreType.DMA,
        pltpu.SemaphoreType.DMA,
    ]

    @functools.partial(
        pl.kernel,
        out_type=jax.ShapeDtypeStruct((NROWS, D), jnp.float32),
        mesh=mesh,
        scratch_types=scratch,
    )
    def scatter(x_hbm, p_hbm, xs_hbm, idx0, idx1, rows0, rows1, sem0, sem1,
                sem_w):
        wid = lax.axis_index("s") * SC_CORES + lax.axis_index("c")
        base = wid * ROWS_PER_W
        idx = (idx0, idx1)
        rows = (rows0, rows1)
        sems = (sem0, sem1)
        pltpu.async_copy(p_hbm.at[pl.ds(base, CH)], idx0, sem0)
        pltpu.async_copy(x_hbm.at[pl.ds(base, CH), :], rows0, sem0)
        for c in range(NCH):
            b = c % 2
            nb = (c + 1) % 2
            if c + 1 < NCH:
                off = base + (c + 1) * CH
                pltpu.async_copy(p_hbm.at[pl.ds(off, CH)], idx[nb], sems[nb])
                pltpu.async_copy(x_hbm.at[pl.ds(off, CH), :], rows[nb],
                                 sems[nb])
            pltpu.make_async_copy(p_hbm.at[pl.ds(base, CH)], idx[b],
                                  sems[b]).wait()
            pltpu.make_async_copy(x_hbm.at[pl.ds(base, CH), :], rows[b],
                                  sems[b]).wait()
            pltpu.async_copy(rows[b], xs_hbm.at[idx[b]], sem_w).wait()

    @functools.partial(
        pl.kernel,
        out_type=jax.ShapeDtypeStruct((N, D), jnp.float32),
        mesh=mesh,
        scratch_types=scratch,
    )
    def gather(ys_hbm, p_hbm, out_hbm, idx0, idx1, rows0, rows1, sem0, sem1,
               sem_w):
        wid = lax.axis_index("s") * SC_CORES + lax.axis_index("c")
        base = wid * ROWS_PER_W
        idx = (idx0, idx1)
        rows = (rows0, rows1)
        sems = (sem0, sem1)
        pltpu.async_copy(p_hbm.at[pl.ds(base, CH)], idx0, sem0)
        for c in range(NCH):
            b = c % 2
            nb = (c + 1) % 2
            if c + 1 < NCH:
                off = base + (c + 1) * CH
                pltpu.async_copy(p_hbm.at[pl.ds(off, CH)], idx[nb], sems[nb])
            pltpu.make_async_copy(p_hbm.at[pl.ds(base, CH)], idx[b],
                                  sems[b]).wait()
            pltpu.async_copy(ys_hbm.at[idx[b]], rows[b], sem_w).wait()
            pltpu.sync_copy(rows[b], out_hbm.at[pl.ds(base + c * CH, CH), :])

    return scatter, gather


def _sc_scatter(x_flat, p_flat):
    return _sc_kernels()[0](x_flat, p_flat)


def _sc_gather(ys, p_flat):
    return _sc_kernels()[1](ys, p_flat)


def _ffn_body(te_ref, ntu_ref, xs_ref, w1_ref, b1_ref, w2_ref, b2_ref,
              ys_ref):
    @pl.when(pl.program_id(0) < ntu_ref[0])
    def _():
        xb = xs_ref[...]
        h = jnp.dot(xb, w1_ref[0], preferred_element_type=jnp.float32)
        h = jnp.maximum(h + b1_ref[0], 0.0)
        y = jnp.dot(h, w2_ref[0], preferred_element_type=jnp.float32)
        ys_ref[...] = y + b2_ref[0]


def _ffn(te, ntu, xs, W1, b1, W2, b2):
    def _clamp(i, ntu):
        return jnp.minimum(i, ntu[0] - 1)

    grid_spec = pltpu.PrefetchScalarGridSpec(
        num_scalar_prefetch=2,
        grid=(NT,),
        in_specs=[
            pl.BlockSpec((T, D), lambda i, te, ntu: (_clamp(i, ntu), 0)),
            pl.BlockSpec((1, D, H),
                         lambda i, te, ntu: (te[_clamp(i, ntu)], 0, 0)),
            pl.BlockSpec((1, 1, H),
                         lambda i, te, ntu: (te[_clamp(i, ntu)], 0, 0)),
            pl.BlockSpec((1, H, D),
                         lambda i, te, ntu: (te[_clamp(i, ntu)], 0, 0)),
            pl.BlockSpec((1, 1, D),
                         lambda i, te, ntu: (te[_clamp(i, ntu)], 0, 0)),
        ],
        out_specs=pl.BlockSpec((T, D), lambda i, te, ntu: (_clamp(i, ntu), 0)),
    )
    return pl.pallas_call(
        _ffn_body,
        grid_spec=grid_spec,
        out_shape=jax.ShapeDtypeStruct((NROWS, D), jnp.float32),
    )(te, ntu, xs, W1, b1.reshape(E, 1, H), W2, b2.reshape(E, 1, D))


def kernel(x, Wg, bg, W1, b1, W2, b2):
    Bb, Ss, _ = x.shape
    x_flat = x.reshape(-1, D)
    gate, ecol, rank, counts = _router(x_flat, Wg, bg)
    p, te, ntu = _plan(counts, gate, rank)
    p_flat = p.reshape(-1)
    xs = _sc_scatter(x_flat, p_flat)
    ys = _ffn(te.reshape(-1), ntu.reshape(-1), xs, W1, b1, W2, b2)
    out = _sc_gather(ys, p_flat)
    return out.reshape(Bb, Ss, D), gate
